# baseline (speedup 1.0000x reference)
import functools

import jax
import jax.numpy as jnp
from jax import lax
from jax.experimental import pallas as pl
from jax.experimental.pallas import tpu as pltpu
from jax.experimental.pallas import tpu_sc as plsc

_B = 4096
_D = 96
_V = 1000001
_NC = 2
_NS = 16
_NW = _NC * _NS
_SCW = 24
_SCN = _SCW * 128
_TCN = _B - _SCN
_BPW = 128
_RB = 256
_L = 16
_NBUF = 8
_DC = _D // _L


def _gather_one_table(tab, idx_v, rows_v, bufs, sems):
    def chunk(g, carry):
        ids16 = idx_v[pl.ds(g * _L, _L)]

        def fire(t):
            sid = ids16[t]
            start = pl.multiple_of((sid // 128) * 128, 128)
            return pltpu.async_copy(
                tab.at[:, pl.ds(start, 128)], bufs[t % _NBUF], sems[t % _NBUF])

        cps = {}
        for t in range(_NBUF - 1):
            cps[t] = fire(t)
        for t in range(_L):
            if t + _NBUF - 1 < _L:
                cps[t + _NBUF - 1] = fire(t + _NBUF - 1)
            cps[t].wait()
            sid = ids16[t]
            r = sid - (sid // 128) * 128
            rvec = jnp.full((_L,), r, jnp.int32)
            jvec = jnp.full((_L,), g * _L + t, jnp.int32)
            buf = bufs[t % _NBUF]
            for c in range(_DC):
                dvec = c * _L + lax.broadcasted_iota(jnp.int32, (_L,), 0)
                vals = plsc.load_gather(buf, [dvec, rvec])
                plsc.store_scatter(rows_v, [dvec, jvec], vals)
        return carry

    lax.fori_loop(0, _BPW // _L, chunk, 0)


def _sc_gather_body(u_tab, b_tab, vids, bids, u_out, b_out,
                    uidx_v, bidx_v, urows_v, brows_v,
                    buf0, buf1, buf2, buf3, buf4, buf5, buf6, buf7,
                    sem0, sem1, sem2, sem3, sem4, sem5, sem6, sem7, osem):
    wid = lax.axis_index("s") * _NC + lax.axis_index("c")

    @pl.when(wid < _SCW)
    def _():
        base = pl.multiple_of(wid * _BPW, _BPW)
        bufs = (buf0, buf1, buf2, buf3, buf4, buf5, buf6, buf7)
        sems = (sem0, sem1, sem2, sem3, sem4, sem5, sem6, sem7)
        pltpu.sync_copy(vids.at[pl.ds(base, _BPW)], uidx_v)
        pltpu.sync_copy(bids.at[pl.ds(base, _BPW)], bidx_v)
        _gather_one_table(u_tab, uidx_v, urows_v, bufs, sems)
        ucp = pltpu.async_copy(urows_v, u_out.at[:, pl.ds(base, _BPW)], osem)
        _gather_one_table(b_tab, bidx_v, brows_v, bufs, sems)
        bcp = pltpu.async_copy(brows_v, b_out.at[:, pl.ds(base, _BPW)], osem)
        ucp.wait()
        bcp.wait()


@functools.lru_cache(maxsize=None)
def _get_sc_gather():
    return pl.kernel(
        _sc_gather_body,
        out_type=(jax.ShapeDtypeStruct((_D, _SCN), jnp.float32),
                  jax.ShapeDtypeStruct((_D, _SCN), jnp.float32)),
        mesh=plsc.VectorSubcoreMesh(core_axis_name="c", subcore_axis_name="s"),
        scratch_types=[
            pltpu.VMEM((_BPW,), jnp.int32),
            pltpu.VMEM((_BPW,), jnp.int32),
            pltpu.VMEM((_D, _BPW), jnp.float32),
            pltpu.VMEM((_D, _BPW), jnp.float32),
            pltpu.VMEM((_D, 128), jnp.float32),
            pltpu.VMEM((_D, 128), jnp.float32),
            pltpu.VMEM((_D, 128), jnp.float32),
            pltpu.VMEM((_D, 128), jnp.float32),
            pltpu.VMEM((_D, 128), jnp.float32),
            pltpu.VMEM((_D, 128), jnp.float32),
            pltpu.VMEM((_D, 128), jnp.float32),
            pltpu.VMEM((_D, 128), jnp.float32),
            pltpu.SemaphoreType.DMA,
            pltpu.SemaphoreType.DMA,
            pltpu.SemaphoreType.DMA,
            pltpu.SemaphoreType.DMA,
            pltpu.SemaphoreType.DMA,
            pltpu.SemaphoreType.DMA,
            pltpu.SemaphoreType.DMA,
            pltpu.SemaphoreType.DMA,
            pltpu.SemaphoreType.DMA,
        ],
        compiler_params=pltpu.CompilerParams(disable_bounds_checks=True,
                                             needs_layout_passes=False),
    )


def _tc_gather_body(idu_ref, idb_ref, utile_ref, btile_ref, uo_ref, bo_ref):
    i = pl.program_id(0)
    lane128 = lax.broadcasted_iota(jnp.int32, (_D, 128), 1)
    tgt = i % 128
    ru = idu_ref[i] % 128
    col_u = jnp.sum(jnp.where(lane128 == ru, utile_ref[...], 0.0), axis=1)
    uo_ref[...] = jnp.where(lane128 == tgt, col_u[:, None], uo_ref[...])
    rb = idb_ref[i] % 128
    col_b = jnp.sum(jnp.where(lane128 == rb, btile_ref[...], 0.0), axis=1)
    bo_ref[...] = jnp.where(lane128 == tgt, col_b[:, None], bo_ref[...])


def _tc_gather(u_tab_t, b_tab_t, idu, idb):
    grid_spec = pltpu.PrefetchScalarGridSpec(
        num_scalar_prefetch=2,
        grid=(_TCN,),
        in_specs=[
            pl.BlockSpec((_D, 128), lambda i, idu, idb: (0, idu[i] // 128)),
            pl.BlockSpec((_D, 128), lambda i, idu, idb: (0, idb[i] // 128)),
        ],
        out_specs=[
            pl.BlockSpec((_D, 128), lambda i, idu, idb: (0, i // 128)),
            pl.BlockSpec((_D, 128), lambda i, idu, idb: (0, i // 128)),
        ],
    )
    return pl.pallas_call(
        _tc_gather_body,
        grid_spec=grid_spec,
        out_shape=(jax.ShapeDtypeStruct((_D, _TCN), jnp.float32),
                   jax.ShapeDtypeStruct((_D, _TCN), jnp.float32)),
        compiler_params=pltpu.CompilerParams(
            dimension_semantics=("arbitrary",)),
    )(idu, idb, u_tab_t, b_tab_t)


def _loss_body(ut_ref, bt_ref, out_ref):
    i = pl.program_id(0)
    ut = ut_ref[...]
    bt = bt_ref[...]
    logits = lax.dot_general(ut.astype(jnp.bfloat16), bt.astype(jnp.bfloat16),
                             (((0,), (0,)), ((), ())),
                             preferred_element_type=jnp.float32)
    m = jnp.max(logits, axis=1, keepdims=True)
    lse = m[:, 0] + jnp.log(jnp.sum(jnp.exp(logits - m), axis=1))
    bt_blk = bt_ref[:, pl.ds(i * _RB, _RB)]
    diag = jnp.sum(ut * bt_blk, axis=0)
    part = jnp.sum(lse - diag)

    @pl.when(i == 0)
    def _():
        out_ref[0, 0] = 0.0

    out_ref[0, 0] += part

    @pl.when(i == _B // _RB - 1)
    def _():
        out_ref[0, 0] = out_ref[0, 0] * (1.0 / _B)


def _loss_sum(ut, bt, interpret=False):
    return pl.pallas_call(
        _loss_body,
        grid=(_B // _RB,),
        in_specs=[
            pl.BlockSpec((_D, _RB), lambda i: (0, i)),
            pl.BlockSpec((_D, _B), lambda i: (0, 0)),
        ],
        out_specs=pl.BlockSpec((1, 1), lambda i: (0, 0),
                               memory_space=pltpu.SMEM),
        out_shape=jax.ShapeDtypeStruct((1, 1), jnp.float32),
        compiler_params=pltpu.CompilerParams(
            dimension_semantics=("arbitrary",)),
        interpret=interpret,
    )(ut, bt)


def kernel(viewer_ids, broadcaster_ids, user_table, broadcaster_table):
    ut_tab = user_table.T
    bt_tab = broadcaster_table.T
    ut_sc, bt_sc = _get_sc_gather()(ut_tab, bt_tab,
                                    viewer_ids[:_SCN], broadcaster_ids[:_SCN])
    ut_tc, bt_tc = _tc_gather(ut_tab, bt_tab,
                              viewer_ids[_SCN:], broadcaster_ids[_SCN:])
    ut = jnp.concatenate([ut_sc, ut_tc], axis=1)
    bt = jnp.concatenate([bt_sc, bt_tc], axis=1)
    total = _loss_sum(ut, bt)
    return total[0, 0]


# speedup vs baseline: 3.0149x; 3.0149x over previous
import functools

import jax
import jax.numpy as jnp
from jax import lax
from jax.experimental import pallas as pl
from jax.experimental.pallas import tpu as pltpu
from jax.experimental.pallas import tpu_sc as plsc

_B = 4096
_D = 96
_V = 1000001
_NC = 2
_NS = 16
_NW = _NC * _NS
_BPW = _B // _NW
_RB = 512
_L = 16
_NBUF = 8
_DC = _D // _L


def _gather_one_table(tab, idx_v, rows_v, bufs, sems):
    def chunk(g, carry):
        ids16 = idx_v[pl.ds(g * _L, _L)]

        def fire(t):
            sid = ids16[t]
            start = pl.multiple_of((sid // 128) * 128, 128)
            return pltpu.async_copy(
                tab.at[:, pl.ds(start, 128)], bufs[t % _NBUF], sems[t % _NBUF])

        cps = {}
        for t in range(_NBUF - 1):
            cps[t] = fire(t)
        for t in range(_L):
            if t + _NBUF - 1 < _L:
                cps[t + _NBUF - 1] = fire(t + _NBUF - 1)
            cps[t].wait()
            sid = ids16[t]
            r = sid - (sid // 128) * 128
            rvec = jnp.full((_L,), r, jnp.int32)
            jvec = jnp.full((_L,), g * _L + t, jnp.int32)
            buf = bufs[t % _NBUF]
            for c in range(_DC):
                dvec = c * _L + lax.broadcasted_iota(jnp.int32, (_L,), 0)
                vals = plsc.load_gather(buf, [dvec, rvec])
                plsc.store_scatter(rows_v, [dvec, jvec], vals)
        return carry

    lax.fori_loop(0, _BPW // _L, chunk, 0)


def _sc_gather_body(u_tab, b_tab, vids, bids, u_out, b_out,
                    uidx_v, bidx_v, urows_v, brows_v,
                    buf0, buf1, buf2, buf3, buf4, buf5, buf6, buf7,
                    sem0, sem1, sem2, sem3, sem4, sem5, sem6, sem7, osem):
    wid = lax.axis_index("s") * _NC + lax.axis_index("c")
    base = pl.multiple_of(wid * _BPW, _BPW)
    bufs = (buf0, buf1, buf2, buf3, buf4, buf5, buf6, buf7)
    sems = (sem0, sem1, sem2, sem3, sem4, sem5, sem6, sem7)
    pltpu.sync_copy(vids.at[pl.ds(base, _BPW)], uidx_v)
    pltpu.sync_copy(bids.at[pl.ds(base, _BPW)], bidx_v)
    _gather_one_table(u_tab, uidx_v, urows_v, bufs, sems)
    ucp = pltpu.async_copy(urows_v, u_out.at[:, pl.ds(base, _BPW)], osem)
    _gather_one_table(b_tab, bidx_v, brows_v, bufs, sems)
    bcp = pltpu.async_copy(brows_v, b_out.at[:, pl.ds(base, _BPW)], osem)
    ucp.wait()
    bcp.wait()


@functools.lru_cache(maxsize=None)
def _get_sc_gather():
    return pl.kernel(
        _sc_gather_body,
        out_type=(jax.ShapeDtypeStruct((_D, _B), jnp.float32),
                  jax.ShapeDtypeStruct((_D, _B), jnp.float32)),
        mesh=plsc.VectorSubcoreMesh(core_axis_name="c", subcore_axis_name="s"),
        scratch_types=[
            pltpu.VMEM((_BPW,), jnp.int32),
            pltpu.VMEM((_BPW,), jnp.int32),
            pltpu.VMEM((_D, _BPW), jnp.float32),
            pltpu.VMEM((_D, _BPW), jnp.float32),
            pltpu.VMEM((_D, 128), jnp.float32),
            pltpu.VMEM((_D, 128), jnp.float32),
            pltpu.VMEM((_D, 128), jnp.float32),
            pltpu.VMEM((_D, 128), jnp.float32),
            pltpu.VMEM((_D, 128), jnp.float32),
            pltpu.VMEM((_D, 128), jnp.float32),
            pltpu.VMEM((_D, 128), jnp.float32),
            pltpu.VMEM((_D, 128), jnp.float32),
            pltpu.SemaphoreType.DMA,
            pltpu.SemaphoreType.DMA,
            pltpu.SemaphoreType.DMA,
            pltpu.SemaphoreType.DMA,
            pltpu.SemaphoreType.DMA,
            pltpu.SemaphoreType.DMA,
            pltpu.SemaphoreType.DMA,
            pltpu.SemaphoreType.DMA,
            pltpu.SemaphoreType.DMA,
        ],
        compiler_params=pltpu.CompilerParams(disable_bounds_checks=True,
                                             needs_layout_passes=False),
    )


def _loss_body(ut_ref, bt_ref, out_ref):
    i = pl.program_id(0)
    ut = ut_ref[...]
    bt = bt_ref[...]
    logits = lax.dot_general(ut.astype(jnp.bfloat16), bt.astype(jnp.bfloat16),
                             (((0,), (0,)), ((), ())),
                             preferred_element_type=jnp.float32)
    m = jnp.max(logits, axis=1, keepdims=True)
    lse = m[:, 0] + jnp.log(jnp.sum(jnp.exp(logits - m), axis=1))
    bt_blk = bt_ref[:, pl.ds(i * _RB, _RB)]
    diag = jnp.sum(ut * bt_blk, axis=0)
    part = jnp.sum(lse - diag)

    @pl.when(i == 0)
    def _():
        out_ref[0, 0] = 0.0

    out_ref[0, 0] += part

    @pl.when(i == _B // _RB - 1)
    def _():
        out_ref[0, 0] = out_ref[0, 0] * (1.0 / _B)


def _loss_sum(ut, bt, interpret=False):
    return pl.pallas_call(
        _loss_body,
        grid=(_B // _RB,),
        in_specs=[
            pl.BlockSpec((_D, _RB), lambda i: (0, i)),
            pl.BlockSpec((_D, _B), lambda i: (0, 0)),
        ],
        out_specs=pl.BlockSpec((1, 1), lambda i: (0, 0),
                               memory_space=pltpu.SMEM),
        out_shape=jax.ShapeDtypeStruct((1, 1), jnp.float32),
        compiler_params=pltpu.CompilerParams(
            dimension_semantics=("arbitrary",)),
        interpret=interpret,
    )(ut, bt)


def kernel(viewer_ids, broadcaster_ids, user_table, broadcaster_table):
    ut, bt = _get_sc_gather()(user_table.T, broadcaster_table.T,
                              viewer_ids, broadcaster_ids)
    total = _loss_sum(ut, bt)
    return total[0, 0]


# speedup vs baseline: 3.0350x; 1.0067x over previous
import functools

import jax
import jax.numpy as jnp
from jax import lax
from jax.experimental import pallas as pl
from jax.experimental.pallas import tpu as pltpu
from jax.experimental.pallas import tpu_sc as plsc

_B = 4096
_D = 96
_V = 1000001
_NC = 2
_NS = 16
_NW = _NC * _NS
_BPW = _B // _NW
_RB = 1024
_L = 16
_NBUF = 8
_DC = _D // _L


def _gather_one_table(tab, idx_v, rows_v, bufs, sems):
    def chunk(g, carry):
        ids16 = idx_v[pl.ds(g * _L, _L)]

        def fire(t):
            sid = ids16[t]
            start = pl.multiple_of((sid // 128) * 128, 128)
            return pltpu.async_copy(
                tab.at[:, pl.ds(start, 128)], bufs[t % _NBUF], sems[t % _NBUF])

        cps = {}
        for t in range(_NBUF - 1):
            cps[t] = fire(t)
        for t in range(_L):
            if t + _NBUF - 1 < _L:
                cps[t + _NBUF - 1] = fire(t + _NBUF - 1)
            cps[t].wait()
            sid = ids16[t]
            r = sid - (sid // 128) * 128
            rvec = jnp.full((_L,), r, jnp.int32)
            jvec = jnp.full((_L,), g * _L + t, jnp.int32)
            buf = bufs[t % _NBUF]
            for c in range(_DC):
                dvec = c * _L + lax.broadcasted_iota(jnp.int32, (_L,), 0)
                vals = plsc.load_gather(buf, [dvec, rvec])
                plsc.store_scatter(rows_v, [dvec, jvec], vals)
        return carry

    lax.fori_loop(0, _BPW // _L, chunk, 0)


def _sc_gather_body(u_tab, b_tab, vids, bids, u_out, b_out,
                    uidx_v, bidx_v, urows_v, brows_v,
                    buf0, buf1, buf2, buf3, buf4, buf5, buf6, buf7,
                    sem0, sem1, sem2, sem3, sem4, sem5, sem6, sem7, osem):
    wid = lax.axis_index("s") * _NC + lax.axis_index("c")
    base = pl.multiple_of(wid * _BPW, _BPW)
    bufs = (buf0, buf1, buf2, buf3, buf4, buf5, buf6, buf7)
    sems = (sem0, sem1, sem2, sem3, sem4, sem5, sem6, sem7)
    pltpu.sync_copy(vids.at[pl.ds(base, _BPW)], uidx_v)
    pltpu.sync_copy(bids.at[pl.ds(base, _BPW)], bidx_v)
    _gather_one_table(u_tab, uidx_v, urows_v, bufs, sems)
    ucp = pltpu.async_copy(urows_v, u_out.at[:, pl.ds(base, _BPW)], osem)
    _gather_one_table(b_tab, bidx_v, brows_v, bufs, sems)
    bcp = pltpu.async_copy(brows_v, b_out.at[:, pl.ds(base, _BPW)], osem)
    ucp.wait()
    bcp.wait()


@functools.lru_cache(maxsize=None)
def _get_sc_gather():
    return pl.kernel(
        _sc_gather_body,
        out_type=(jax.ShapeDtypeStruct((_D, _B), jnp.float32),
                  jax.ShapeDtypeStruct((_D, _B), jnp.float32)),
        mesh=plsc.VectorSubcoreMesh(core_axis_name="c", subcore_axis_name="s"),
        scratch_types=[
            pltpu.VMEM((_BPW,), jnp.int32),
            pltpu.VMEM((_BPW,), jnp.int32),
            pltpu.VMEM((_D, _BPW), jnp.float32),
            pltpu.VMEM((_D, _BPW), jnp.float32),
            pltpu.VMEM((_D, 128), jnp.float32),
            pltpu.VMEM((_D, 128), jnp.float32),
            pltpu.VMEM((_D, 128), jnp.float32),
            pltpu.VMEM((_D, 128), jnp.float32),
            pltpu.VMEM((_D, 128), jnp.float32),
            pltpu.VMEM((_D, 128), jnp.float32),
            pltpu.VMEM((_D, 128), jnp.float32),
            pltpu.VMEM((_D, 128), jnp.float32),
            pltpu.SemaphoreType.DMA,
            pltpu.SemaphoreType.DMA,
            pltpu.SemaphoreType.DMA,
            pltpu.SemaphoreType.DMA,
            pltpu.SemaphoreType.DMA,
            pltpu.SemaphoreType.DMA,
            pltpu.SemaphoreType.DMA,
            pltpu.SemaphoreType.DMA,
            pltpu.SemaphoreType.DMA,
        ],
        compiler_params=pltpu.CompilerParams(disable_bounds_checks=True,
                                             needs_layout_passes=False),
    )


def _loss_body(ut_ref, bt_ref, out_ref):
    i = pl.program_id(0)
    ut = ut_ref[...]
    bt = bt_ref[...]
    logits = lax.dot_general(ut.astype(jnp.bfloat16), bt.astype(jnp.bfloat16),
                             (((0,), (0,)), ((), ())),
                             preferred_element_type=jnp.float32)
    m = jnp.max(logits, axis=1, keepdims=True)
    lse = m[:, 0] + jnp.log(jnp.sum(jnp.exp(logits - m), axis=1))
    bt_blk = bt_ref[:, pl.ds(i * _RB, _RB)]
    diag = jnp.sum(ut * bt_blk, axis=0)
    part = jnp.sum(lse - diag)

    @pl.when(i == 0)
    def _():
        out_ref[0, 0] = 0.0

    out_ref[0, 0] += part

    @pl.when(i == _B // _RB - 1)
    def _():
        out_ref[0, 0] = out_ref[0, 0] * (1.0 / _B)


def _loss_sum(ut, bt, interpret=False):
    return pl.pallas_call(
        _loss_body,
        grid=(_B // _RB,),
        in_specs=[
            pl.BlockSpec((_D, _RB), lambda i: (0, i)),
            pl.BlockSpec((_D, _B), lambda i: (0, 0)),
        ],
        out_specs=pl.BlockSpec((1, 1), lambda i: (0, 0),
                               memory_space=pltpu.SMEM),
        out_shape=jax.ShapeDtypeStruct((1, 1), jnp.float32),
        compiler_params=pltpu.CompilerParams(
            dimension_semantics=("arbitrary",)),
        interpret=interpret,
    )(ut, bt)


def kernel(viewer_ids, broadcaster_ids, user_table, broadcaster_table):
    ut, bt = _get_sc_gather()(user_table.T, broadcaster_table.T,
                              viewer_ids, broadcaster_ids)
    total = _loss_sum(ut, bt)
    return total[0, 0]
